# Initial kernel scaffold; baseline (speedup 1.0000x reference)
import jax
import jax.numpy as jnp
from jax.experimental import pallas as pl


def kernel(x, edge_index, W1l, b1, W1r, W2l, b2, W2r):
    raise NotImplementedError("write your pallas kernel here")


# speedup vs baseline: 7.4206x; 7.4206x over previous
import functools

import jax
import jax.numpy as jnp
from jax import lax
from jax.experimental import pallas as pl
from jax.experimental.pallas import tpu as pltpu
from jax.experimental.pallas import tpu_sc as plsc

N = 10000
E = 160000
D = 256
H = 256
NCLS = 2

NC = 2
NS = 16
HALF = 144
W2PAD = 16
RB = 1000
GRID = N // RB

K1 = 40
NB1 = (E // NS) // K1
K2 = 40
NB2 = (E // (NC * NS)) // K2
RPT = N // NS


def _tc_a_body(x_ref, w1l_ref, w1r_ref, b1_ref, paug_ref, r_ref):
    xb = x_ref[...]
    p = lax.dot_general(xb, w1l_ref[...], (((1,), (1,)), ((), ())),
                        preferred_element_type=jnp.float32)
    r = lax.dot_general(xb, w1r_ref[...], (((1,), (1,)), ((), ())),
                        preferred_element_type=jnp.float32) + b1_ref[...]
    ones = jnp.ones((RB, 1), jnp.float32)
    zeros = jnp.zeros((RB, 2 * HALF - D - 1), jnp.float32)
    paug_ref[0] = p[:, :HALF]
    paug_ref[1] = jnp.concatenate([p[:, HALF:], ones, zeros], axis=1)
    r_ref[...] = r


_tc_a = pl.pallas_call(
    _tc_a_body,
    grid=(GRID,),
    in_specs=[
        pl.BlockSpec((RB, D), lambda i: (i, 0)),
        pl.BlockSpec((H, D), lambda i: (0, 0)),
        pl.BlockSpec((H, D), lambda i: (0, 0)),
        pl.BlockSpec((1, H), lambda i: (0, 0)),
    ],
    out_specs=[
        pl.BlockSpec((NC, RB, HALF), lambda i: (0, i, 0)),
        pl.BlockSpec((RB, H), lambda i: (i, 0)),
    ],
    out_shape=[
        jax.ShapeDtypeStruct((NC, N, HALF), jnp.float32),
        jax.ShapeDtypeStruct((N, H), jnp.float32),
    ],
)


@functools.lru_cache(maxsize=None)
def _make_seg_sum(table_rows, width, nb, k):
    def body(table_ref, src_ref, dst_ref, out_ref,
             src2d, dst2d, rows0, rows1, acc, sem0, sem1):
        c = lax.axis_index("c")
        s = lax.axis_index("s")
        pltpu.sync_copy(src_ref.at[c, s], src2d)
        pltpu.sync_copy(dst_ref.at[c, s], dst2d)

        nlane = width // 16

        def _z(i, carry):
            r = i // nlane
            j = i % nlane
            rows0[r, pl.ds(j * 16, 16)] = jnp.zeros((16,), jnp.float32)
            return carry

        lax.fori_loop(0, k * nlane, _z, 0)
        nfull = RPT // k
        rem = RPT - nfull * k
        for q in range(nfull):
            pltpu.sync_copy(rows0, acc.at[pl.ds(s * RPT + q * k, k)])
        if rem:
            pltpu.sync_copy(rows0.at[pl.ds(0, rem)],
                            acc.at[pl.ds(s * RPT + nfull * k, rem)])
        plsc.subcore_barrier()

        pltpu.async_copy(table_ref.at[src2d.at[0]], rows0, sem0)

        def _blk(b, carry):
            nxt = b + 1

            @pl.when(jnp.logical_and(nxt < nb, nxt % 2 == 1))
            def _():
                pltpu.async_copy(table_ref.at[src2d.at[nxt]], rows1, sem1)

            @pl.when(jnp.logical_and(nxt < nb, nxt % 2 == 0))
            def _():
                pltpu.async_copy(table_ref.at[src2d.at[nxt]], rows0, sem0)

            @pl.when(b % 2 == 0)
            def _():
                pltpu.make_async_copy(
                    table_ref.at[src2d.at[b]], rows0, sem0).wait()
                pltpu.sync_copy(rows0, acc.at[dst2d.at[b]], add=True)

            @pl.when(b % 2 == 1)
            def _():
                pltpu.make_async_copy(
                    table_ref.at[src2d.at[b]], rows1, sem1).wait()
                pltpu.sync_copy(rows1, acc.at[dst2d.at[b]], add=True)

            return carry

        lax.fori_loop(0, nb, _blk, 0)
        plsc.subcore_barrier()
        pltpu.sync_copy(acc.at[pl.ds(s * RPT, RPT)],
                        out_ref.at[c, pl.ds(s * RPT, RPT)])

    return pl.kernel(
        body,
        out_type=jax.ShapeDtypeStruct((NC, N, width), jnp.float32),
        mesh=plsc.VectorSubcoreMesh(core_axis_name="c", subcore_axis_name="s"),
        scratch_types=[
            pltpu.VMEM((nb, k), jnp.int32),
            pltpu.VMEM((nb, k), jnp.int32),
            pltpu.VMEM((k, width), jnp.float32),
            pltpu.VMEM((k, width), jnp.float32),
            pltpu.VMEM_SHARED((N, width), jnp.float32),
            pltpu.SemaphoreType.DMA,
            pltpu.SemaphoreType.DMA,
        ],
        compiler_params=pltpu.CompilerParams(use_tc_tiling_on_sc=False),
    )


def _tc_b_body(seg_ref, r_ref, w2l_ref, w2r_ref, b2_ref, qaug_ref, r2c_ref):
    seg_a = seg_ref[0]
    seg_b = seg_ref[1]
    sums = jnp.concatenate([seg_a, seg_b[:, :D - HALF]], axis=1)
    denom = jnp.maximum(seg_b[:, D - HALF:D - HALF + 1], 1.0)
    h = jnp.maximum(sums / denom + r_ref[...], 0.0)
    q = lax.dot_general(h, w2l_ref[...], (((1,), (1,)), ((), ())),
                        preferred_element_type=jnp.float32)
    r2 = lax.dot_general(h, w2r_ref[...], (((1,), (1,)), ((), ())),
                         preferred_element_type=jnp.float32) + b2_ref[...]
    qaug_ref[...] = q
    r2c_ref[...] = jnp.concatenate(
        [r2[:, :NCLS], denom, jnp.zeros((RB, W2PAD - NCLS - 1), jnp.float32)],
        axis=1)


_tc_b = pl.pallas_call(
    _tc_b_body,
    grid=(GRID,),
    in_specs=[
        pl.BlockSpec((NC, RB, HALF), lambda i: (0, i, 0)),
        pl.BlockSpec((RB, H), lambda i: (i, 0)),
        pl.BlockSpec((W2PAD, H), lambda i: (0, 0)),
        pl.BlockSpec((W2PAD, H), lambda i: (0, 0)),
        pl.BlockSpec((1, W2PAD), lambda i: (0, 0)),
    ],
    out_specs=[
        pl.BlockSpec((RB, W2PAD), lambda i: (i, 0)),
        pl.BlockSpec((RB, W2PAD), lambda i: (i, 0)),
    ],
    out_shape=[
        jax.ShapeDtypeStruct((N, W2PAD), jnp.float32),
        jax.ShapeDtypeStruct((N, W2PAD), jnp.float32),
    ],
)


def _tc_c_body(seg2_ref, r2c_ref, out_ref):
    s2 = seg2_ref[0] + seg2_ref[1]
    r2c = r2c_ref[...]
    out_ref[...] = s2[:, :NCLS] / r2c[:, NCLS:NCLS + 1] + r2c[:, :NCLS]


_tc_c = pl.pallas_call(
    _tc_c_body,
    grid=(GRID,),
    in_specs=[
        pl.BlockSpec((NC, RB, W2PAD), lambda i: (0, i, 0)),
        pl.BlockSpec((RB, W2PAD), lambda i: (i, 0)),
    ],
    out_specs=pl.BlockSpec((RB, NCLS), lambda i: (i, 0)),
    out_shape=jax.ShapeDtypeStruct((N, NCLS), jnp.float32),
)


def kernel(x, edge_index, W1l, b1, W1r, W2l, b2, W2r):
    src = edge_index[0].astype(jnp.int32)
    dst = edge_index[1].astype(jnp.int32)

    paug, r1 = _tc_a(x, W1l, W1r, b1.reshape(1, H))
    table1 = paug.reshape(NC * N, HALF)

    srcp1 = jnp.stack([src, src + N]).reshape(NC, NS, NB1, K1)
    dst1 = jnp.stack([dst, dst]).reshape(NC, NS, NB1, K1)
    seg1 = _make_seg_sum(NC * N, HALF, NB1, K1)(table1, srcp1, dst1)

    w2lp = jnp.zeros((W2PAD, H), jnp.float32).at[:NCLS].set(W2l)
    w2rp = jnp.zeros((W2PAD, H), jnp.float32).at[:NCLS].set(W2r)
    b2p = jnp.zeros((1, W2PAD), jnp.float32).at[0, :NCLS].set(b2)
    qaug, r2c = _tc_b(seg1, r1, w2lp, w2rp, b2p)

    src2 = src.reshape(NC, NS, NB2, K2)
    dst2 = dst.reshape(NC, NS, NB2, K2)
    seg2 = _make_seg_sum(N, W2PAD, NB2, K2)(qaug, src2, dst2)

    return _tc_c(seg2, r2c)


# speedup vs baseline: 8.3532x; 1.1257x over previous
import functools

import jax
import jax.numpy as jnp
from jax import lax
from jax.experimental import pallas as pl
from jax.experimental.pallas import tpu as pltpu
from jax.experimental.pallas import tpu_sc as plsc

N = 10000
E = 160000
D = 256
H = 256
NCLS = 2

NC = 2
NS = 16
HALF = 144
W2PAD = 16
RB = 1000
GRID = N // RB

K1 = 80
NB1 = (E // NS) // K1
K2 = 40
NB2 = (E // (NC * NS)) // K2
NCH = 5
RPT = N // NS


def _tc_a_body(x_ref, w1l_ref, w1r_ref, b1_ref, paug_ref, r_ref):
    xb = x_ref[...]
    p = lax.dot_general(xb, w1l_ref[...], (((1,), (1,)), ((), ())),
                        preferred_element_type=jnp.float32)
    r = lax.dot_general(xb, w1r_ref[...], (((1,), (1,)), ((), ())),
                        preferred_element_type=jnp.float32) + b1_ref[...]
    ones = jnp.ones((RB, 1), jnp.float32)
    zeros = jnp.zeros((RB, 2 * HALF - D - 1), jnp.float32)
    paug_ref[0] = p[:, :HALF]
    paug_ref[1] = jnp.concatenate([p[:, HALF:], ones, zeros], axis=1)
    r_ref[...] = r


_tc_a = pl.pallas_call(
    _tc_a_body,
    grid=(GRID,),
    in_specs=[
        pl.BlockSpec((RB, D), lambda i: (i, 0)),
        pl.BlockSpec((H, D), lambda i: (0, 0)),
        pl.BlockSpec((H, D), lambda i: (0, 0)),
        pl.BlockSpec((1, H), lambda i: (0, 0)),
    ],
    out_specs=[
        pl.BlockSpec((NC, RB, HALF), lambda i: (0, i, 0)),
        pl.BlockSpec((RB, H), lambda i: (i, 0)),
    ],
    out_shape=[
        jax.ShapeDtypeStruct((NC, N, HALF), jnp.float32),
        jax.ShapeDtypeStruct((N, H), jnp.float32),
    ],
)


@functools.lru_cache(maxsize=None)
def _make_seg_sum(table_rows, width, nb, k):
    cb = nb // NCH

    def body(table_ref, src_ref, dst_ref, out_ref,
             src_a, src_b, dst_a, dst_b, rows0, rows1, acc,
             semi_a, semi_b, semg0, semg1, sems0, sems1):
        c = lax.axis_index("c")
        s = lax.axis_index("s")

        nlane = width // 16

        def _z(i, carry):
            r = i // nlane
            j = i % nlane
            rows0[r, pl.ds(j * 16, 16)] = jnp.zeros((16,), jnp.float32)
            return carry

        lax.fori_loop(0, k * nlane, _z, 0)
        nfull = RPT // k
        rem = RPT - nfull * k
        for q in range(nfull):
            pltpu.sync_copy(rows0, acc.at[pl.ds(s * RPT + q * k, k)])
        if rem:
            pltpu.sync_copy(rows0.at[pl.ds(0, rem)],
                            acc.at[pl.ds(s * RPT + nfull * k, rem)])

        pltpu.sync_copy(src_ref.at[c, s, pl.ds(0, cb)], src_a)
        pltpu.sync_copy(dst_ref.at[c, s, pl.ds(0, cb)], dst_a)
        plsc.subcore_barrier()

        bufs = [(src_a, dst_a, semi_a), (src_b, dst_b, semi_b)]
        for ch in range(NCH):
            sbuf, dbuf, semi = bufs[ch % 2]
            sbuf_n, dbuf_n, semi_n = bufs[(ch + 1) % 2]
            if ch > 0:
                pltpu.make_async_copy(
                    src_ref.at[c, s, pl.ds(ch * cb, cb)], sbuf, semi).wait()
                pltpu.make_async_copy(
                    dst_ref.at[c, s, pl.ds(ch * cb, cb)], dbuf, semi).wait()
            if ch + 1 < NCH:
                pltpu.async_copy(
                    src_ref.at[c, s, pl.ds((ch + 1) * cb, cb)], sbuf_n, semi_n)
                pltpu.async_copy(
                    dst_ref.at[c, s, pl.ds((ch + 1) * cb, cb)], dbuf_n, semi_n)

            pltpu.async_copy(table_ref.at[sbuf.at[0]], rows0, semg0)

            def _blk(b, carry, sbuf=sbuf, dbuf=dbuf):
                @pl.when(jnp.logical_and(b + 1 < cb, (b + 1) % 2 == 0))
                def _():
                    pltpu.async_copy(table_ref.at[sbuf.at[b + 1]], rows0, semg0)

                @pl.when(jnp.logical_and(b + 1 < cb, (b + 1) % 2 == 1))
                def _():
                    pltpu.async_copy(table_ref.at[sbuf.at[b + 1]], rows1, semg1)

                @pl.when(b % 2 == 0)
                def _():
                    pltpu.make_async_copy(
                        table_ref.at[sbuf.at[b]], rows0, semg0).wait()
                    pltpu.sync_copy(rows0, acc.at[dbuf.at[b]], add=True)

                @pl.when(b % 2 == 1)
                def _():
                    pltpu.make_async_copy(
                        table_ref.at[sbuf.at[b]], rows1, semg1).wait()
                    pltpu.sync_copy(rows1, acc.at[dbuf.at[b]], add=True)

                return carry

            lax.fori_loop(0, cb, _blk, 0)

        plsc.subcore_barrier()
        pltpu.sync_copy(acc.at[pl.ds(s * RPT, RPT)],
                        out_ref.at[c, pl.ds(s * RPT, RPT)])

    return pl.kernel(
        body,
        out_type=jax.ShapeDtypeStruct((NC, N, width), jnp.float32),
        mesh=plsc.VectorSubcoreMesh(core_axis_name="c", subcore_axis_name="s"),
        scratch_types=[
            pltpu.VMEM((cb, k), jnp.int32),
            pltpu.VMEM((cb, k), jnp.int32),
            pltpu.VMEM((cb, k), jnp.int32),
            pltpu.VMEM((cb, k), jnp.int32),
            pltpu.VMEM((k, width), jnp.float32),
            pltpu.VMEM((k, width), jnp.float32),
            pltpu.VMEM_SHARED((N, width), jnp.float32),
            pltpu.SemaphoreType.DMA,
            pltpu.SemaphoreType.DMA,
            pltpu.SemaphoreType.DMA,
            pltpu.SemaphoreType.DMA,
            pltpu.SemaphoreType.DMA,
            pltpu.SemaphoreType.DMA,
        ],
        compiler_params=pltpu.CompilerParams(use_tc_tiling_on_sc=False),
    )


def _tc_b_body(seg_ref, r_ref, w2l_ref, w2r_ref, b2_ref, qaug_ref, r2c_ref):
    seg_a = seg_ref[0]
    seg_b = seg_ref[1]
    sums = jnp.concatenate([seg_a, seg_b[:, :D - HALF]], axis=1)
    denom = jnp.maximum(seg_b[:, D - HALF:D - HALF + 1], 1.0)
    h = jnp.maximum(sums / denom + r_ref[...], 0.0)
    q = lax.dot_general(h, w2l_ref[...], (((1,), (1,)), ((), ())),
                        preferred_element_type=jnp.float32)
    r2 = lax.dot_general(h, w2r_ref[...], (((1,), (1,)), ((), ())),
                         preferred_element_type=jnp.float32) + b2_ref[...]
    qaug_ref[...] = q
    r2c_ref[...] = jnp.concatenate(
        [r2[:, :NCLS], denom, jnp.zeros((RB, W2PAD - NCLS - 1), jnp.float32)],
        axis=1)


_tc_b = pl.pallas_call(
    _tc_b_body,
    grid=(GRID,),
    in_specs=[
        pl.BlockSpec((NC, RB, HALF), lambda i: (0, i, 0)),
        pl.BlockSpec((RB, H), lambda i: (i, 0)),
        pl.BlockSpec((W2PAD, H), lambda i: (0, 0)),
        pl.BlockSpec((W2PAD, H), lambda i: (0, 0)),
        pl.BlockSpec((1, W2PAD), lambda i: (0, 0)),
    ],
    out_specs=[
        pl.BlockSpec((RB, W2PAD), lambda i: (i, 0)),
        pl.BlockSpec((RB, W2PAD), lambda i: (i, 0)),
    ],
    out_shape=[
        jax.ShapeDtypeStruct((N, W2PAD), jnp.float32),
        jax.ShapeDtypeStruct((N, W2PAD), jnp.float32),
    ],
)


def _tc_c_body(seg2_ref, r2c_ref, out_ref):
    s2 = seg2_ref[0] + seg2_ref[1]
    r2c = r2c_ref[...]
    out_ref[...] = s2[:, :NCLS] / r2c[:, NCLS:NCLS + 1] + r2c[:, :NCLS]


_tc_c = pl.pallas_call(
    _tc_c_body,
    grid=(GRID,),
    in_specs=[
        pl.BlockSpec((NC, RB, W2PAD), lambda i: (0, i, 0)),
        pl.BlockSpec((RB, W2PAD), lambda i: (i, 0)),
    ],
    out_specs=pl.BlockSpec((RB, NCLS), lambda i: (i, 0)),
    out_shape=jax.ShapeDtypeStruct((N, NCLS), jnp.float32),
)


def kernel(x, edge_index, W1l, b1, W1r, W2l, b2, W2r):
    src = edge_index[0].astype(jnp.int32)
    dst = edge_index[1].astype(jnp.int32)

    paug, r1 = _tc_a(x, W1l, W1r, b1.reshape(1, H))
    table1 = paug.reshape(NC * N, HALF)

    srcp1 = jnp.stack([src, src + N]).reshape(NC, NS, NB1, K1)
    dst1 = jnp.stack([dst, dst]).reshape(NC, NS, NB1, K1)
    seg1 = _make_seg_sum(NC * N, HALF, NB1, K1)(table1, srcp1, dst1)

    w2lp = jnp.zeros((W2PAD, H), jnp.float32).at[:NCLS].set(W2l)
    w2rp = jnp.zeros((W2PAD, H), jnp.float32).at[:NCLS].set(W2r)
    b2p = jnp.zeros((1, W2PAD), jnp.float32).at[0, :NCLS].set(b2)
    qaug, r2c = _tc_b(seg1, r1, w2lp, w2rp, b2p)

    src2 = src.reshape(NC, NS, NB2, K2)
    dst2 = dst.reshape(NC, NS, NB2, K2)
    seg2 = _make_seg_sum(N, W2PAD, NB2, K2)(qaug, src2, dst2)

    return _tc_c(seg2, r2c)
